# Initial kernel scaffold; baseline (speedup 1.0000x reference)
import jax
import jax.numpy as jnp
from jax.experimental import pallas as pl


def kernel(queries, query_geometry_logits, value, W_off, b_off, W_attn, b_attn, W_val, b_val, W_final, b_final):
    raise NotImplementedError("write your pallas kernel here")


# speedup vs baseline: 11.6390x; 11.6390x over previous
import functools

import jax
import jax.numpy as jnp
import numpy as np
from jax import lax
from jax.experimental import pallas as pl
from jax.experimental.pallas import tpu as pltpu
from jax.experimental.pallas import tpu_sc as plsc

B, Q, D = 2, 1000, 256
H = 8
L = 4
K = 4
LK = L * K
Cv = 256
Hs, Ws = 100, 100
SCALE = 0.5
P = Hs * Ws
HLK = H * LK
NOUT = B * Q * H
NW = 32
PER_W = NOUT // NW


def _stage_a_body(q_ref, qgl_ref, wattn_ref, battn_ref, woffx_ref, boffx_ref,
                  woffy_ref, boffy_ref, g_ref,
                  i00_ref, i01_ref, i10_ref, i11_ref,
                  w00_ref, w01_ref, w10_ref, w11_ref):
    b = pl.program_id(0)
    q = q_ref[0]

    attn_log = jnp.dot(q, wattn_ref[...], preferred_element_type=jnp.float32)
    attn_log = attn_log + battn_ref[...]
    m = jnp.max(attn_log, axis=-1, keepdims=True)
    e = jnp.exp(attn_log - m)
    s = jnp.dot(e, g_ref[...], preferred_element_type=jnp.float32)
    attn = e / s

    ox = jnp.dot(q, woffx_ref[...], preferred_element_type=jnp.float32) + boffx_ref[...]
    oy = jnp.dot(q, woffy_ref[...], preferred_element_type=jnp.float32) + boffy_ref[...]

    qgl = qgl_ref[0]
    box = 1.0 / (1.0 + jnp.exp(-qgl))
    cx = box[:, 0:1]
    cy = box[:, 1:2]
    sx = cx * (SCALE / LK)
    sy = cy * (SCALE / LK)

    locx = cx + ox * sx
    locy = cy + oy * sy
    gxn = jnp.clip(2.0 * locx - 1.0, -1.0, 1.0)
    gyn = jnp.clip(2.0 * locy - 1.0, -1.0, 1.0)
    gx = ((gxn + 1.0) * Ws - 1.0) * 0.5
    gy = ((gyn + 1.0) * Hs - 1.0) * 0.5

    x0f = jnp.floor(gx)
    y0f = jnp.floor(gy)
    wx1 = gx - x0f
    wx0 = 1.0 - wx1
    wy1 = gy - y0f
    wy0 = 1.0 - wy1
    x0 = x0f.astype(jnp.int32)
    y0 = y0f.astype(jnp.int32)
    x1 = x0 + 1
    y1 = y0 + 1

    boff = b * P

    def corner(xc, yc, wxc, wyc):
        valid = ((xc >= 0) & (xc <= Ws - 1) & (yc >= 0) & (yc <= Hs - 1))
        idx = (jnp.clip(yc, 0, Hs - 1) * Ws + jnp.clip(xc, 0, Ws - 1) + boff)
        w = wxc * wyc * valid.astype(jnp.float32) * attn
        return idx, w

    i00, w00 = corner(x0, y0, wx0, wy0)
    i01, w01 = corner(x0, y1, wx0, wy1)
    i10, w10 = corner(x1, y0, wx1, wy0)
    i11, w11 = corner(x1, y1, wx1, wy1)

    i00_ref[0] = i00
    i01_ref[0] = i01
    i10_ref[0] = i10
    i11_ref[0] = i11
    w00_ref[0] = w00
    w01_ref[0] = w01
    w10_ref[0] = w10
    w11_ref[0] = w11


def _stage_a(queries, qgl, W_attn, b_attn, W_off_x, b_off_x, W_off_y, b_off_y, G):
    out_sd = jax.ShapeDtypeStruct((B, Q, HLK), jnp.int32)
    out_sw = jax.ShapeDtypeStruct((B, Q, HLK), jnp.float32)
    full = lambda shp: pl.BlockSpec(shp, lambda b: (0,) * len(shp))
    bq = pl.BlockSpec((1, Q, D), lambda b: (b, 0, 0))
    bg = pl.BlockSpec((1, Q, 4), lambda b: (b, 0, 0))
    bo = pl.BlockSpec((1, Q, HLK), lambda b: (b, 0, 0))
    return pl.pallas_call(
        _stage_a_body,
        grid=(B,),
        in_specs=[bq, bg, full((D, HLK)), full((1, HLK)), full((D, HLK)),
                  full((1, HLK)), full((D, HLK)), full((1, HLK)), full((HLK, HLK))],
        out_specs=[bo] * 8,
        out_shape=[out_sd] * 4 + [out_sw] * 4,
    )(queries, qgl, W_attn, b_attn, W_off_x, b_off_x, W_off_y, b_off_y, G)


def _vproj_body(vt_ref, w_ref, b_ref, out_ref):
    out_ref[...] = (jnp.dot(vt_ref[...], w_ref[...],
                            preferred_element_type=jnp.float32) + b_ref[...])


def _vproj(value_t, W_val, b_val):
    nb = 10
    rows = (B * P) // nb
    return pl.pallas_call(
        _vproj_body,
        grid=(nb,),
        in_specs=[pl.BlockSpec((rows, D), lambda i: (i, 0)),
                  pl.BlockSpec((D, Cv), lambda i: (0, 0)),
                  pl.BlockSpec((1, Cv), lambda i: (0, 0))],
        out_specs=pl.BlockSpec((rows, Cv), lambda i: (i, 0)),
        out_shape=jax.ShapeDtypeStruct((B * P, Cv), jnp.float32),
    )(value_t, W_val, b_val)


def _sc_gather_body(table, idx, w, out, idx_v, w_v, rows_v, acc_v, sem):
    nc = 2
    wid = lax.axis_index("s") * nc + lax.axis_index("c")
    base = wid * PER_W

    def body(i, _):
        r = base + i
        pltpu.sync_copy(idx.at[r], idx_v)
        pltpu.sync_copy(w.at[r], w_v)
        pltpu.async_copy(table.at[idx_v], rows_v, sem).wait()

        def jbody(jj, accs):
            wchunk = w_v[pl.ds(jj * 16, 16)]
            for t in range(16):
                wv = lax.gather(
                    wchunk, jnp.full((16, 1), t, jnp.int32),
                    lax.GatherDimensionNumbers(offset_dims=(),
                                               collapsed_slice_dims=(0,),
                                               start_index_map=(0,)),
                    (1,), mode=lax.GatherScatterMode.PROMISE_IN_BOUNDS)
                j = jj * 16 + t
                accs = tuple(accs[c] + wv * rows_v[j, pl.ds(c * 16, 16)]
                             for c in range(16))
            return accs

        accs = lax.fori_loop(
            0, (LK * 4) // 16, jbody,
            tuple(jnp.zeros((16,), jnp.float32) for _ in range(16)))
        for c in range(16):
            acc_v[pl.ds(c * 16, 16)] = accs[c]
        pltpu.sync_copy(acc_v, out.at[r])
        return 0

    lax.fori_loop(0, PER_W, body, 0)


def _sc_gather(table, idx_rows, w_rows):
    mesh = plsc.VectorSubcoreMesh(core_axis_name="c", subcore_axis_name="s")
    kfn = functools.partial(
        pl.kernel, mesh=mesh,
        out_type=jax.ShapeDtypeStruct((NOUT, Cv), jnp.float32),
        scratch_types=[
            pltpu.VMEM((LK * 4,), jnp.int32),
            pltpu.VMEM((LK * 4,), jnp.float32),
            pltpu.VMEM((LK * 4, Cv), jnp.float32),
            pltpu.VMEM((Cv,), jnp.float32),
            pltpu.SemaphoreType.DMA,
        ],
    )(_sc_gather_body)
    return kfn(table, idx_rows, w_rows)


def _final_body(hm_ref, wf_ref, bf_ref, out_ref):
    acc = jnp.dot(hm_ref[...], wf_ref[...], preferred_element_type=jnp.float32)
    out_ref[...] = acc + jnp.sum(bf_ref[...], axis=0, keepdims=True)


def _final(hm, Wf, b_final):
    return pl.pallas_call(
        _final_body,
        grid=(2,),
        in_specs=[pl.BlockSpec((B * Q // 2, H * Cv), lambda i: (i, 0)),
                  pl.BlockSpec((H * Cv, Cv), lambda i: (0, 0)),
                  pl.BlockSpec((H, Cv), lambda i: (0, 0))],
        out_specs=pl.BlockSpec((B * Q // 2, Cv), lambda i: (i, 0)),
        out_shape=jax.ShapeDtypeStruct((B * Q, Cv), jnp.float32),
    )(hm, Wf, b_final)


_G_BLOCK = np.kron(np.eye(H, dtype=np.float32), np.ones((LK, LK), np.float32))


def kernel(queries, query_geometry_logits, value, W_off, b_off, W_attn, b_attn,
           W_val, b_val, W_final, b_final):
    W_off_x = W_off[:, 0::2]
    W_off_y = W_off[:, 1::2]
    b_off_x = b_off[0::2].reshape(1, HLK)
    b_off_y = b_off[1::2].reshape(1, HLK)
    b_attn2 = b_attn.reshape(1, HLK)
    G = jnp.asarray(_G_BLOCK)

    i00, i01, i10, i11, w00, w01, w10, w11 = _stage_a(
        queries, query_geometry_logits, W_attn, b_attn2,
        W_off_x, b_off_x, W_off_y, b_off_y, G)

    idx_rows = jnp.stack([i00, i01, i10, i11], axis=-1).reshape(NOUT, LK * 4)
    w_rows = jnp.stack([w00, w01, w10, w11], axis=-1).reshape(NOUT, LK * 4)

    value_t = value.transpose(0, 2, 3, 1).reshape(B * P, D)
    table = _vproj(value_t, W_val, b_val.reshape(1, Cv))

    heads = _sc_gather(table, idx_rows, w_rows)

    hm = heads.reshape(B * Q, H * Cv)
    Wf = W_final.reshape(H * Cv, Cv)
    out = _final(hm, Wf, b_final)
    return out.reshape(B, Q, Cv)


# speedup vs baseline: 12.4920x; 1.0733x over previous
import functools

import jax
import jax.numpy as jnp
import numpy as np
from jax import lax
from jax.experimental import pallas as pl
from jax.experimental.pallas import tpu as pltpu
from jax.experimental.pallas import tpu_sc as plsc

B, Q, D = 2, 1000, 256
H = 8
L = 4
K = 4
LK = L * K
Cv = 256
Hs, Ws = 100, 100
SCALE = 0.5
P = Hs * Ws
HLK = H * LK
NOUT = B * Q * H
NW = 32
NOUT_PAD = 16384
PER_W = NOUT_PAD // NW


def _stage_a_body(q_ref, qgl_ref, wattn_ref, battn_ref, woffx_ref, boffx_ref,
                  woffy_ref, boffy_ref, g_ref,
                  i00_ref, i01_ref, i10_ref, i11_ref,
                  w00_ref, w01_ref, w10_ref, w11_ref):
    b = pl.program_id(0)
    q = q_ref[0]

    attn_log = jnp.dot(q, wattn_ref[...], preferred_element_type=jnp.float32)
    attn_log = attn_log + battn_ref[...]
    m = jnp.max(attn_log, axis=-1, keepdims=True)
    e = jnp.exp(attn_log - m)
    s = jnp.dot(e, g_ref[...], preferred_element_type=jnp.float32)
    attn = e / s

    ox = jnp.dot(q, woffx_ref[...], preferred_element_type=jnp.float32) + boffx_ref[...]
    oy = jnp.dot(q, woffy_ref[...], preferred_element_type=jnp.float32) + boffy_ref[...]

    qgl = qgl_ref[0]
    box = 1.0 / (1.0 + jnp.exp(-qgl))
    cx = box[:, 0:1]
    cy = box[:, 1:2]
    sx = cx * (SCALE / LK)
    sy = cy * (SCALE / LK)

    locx = cx + ox * sx
    locy = cy + oy * sy
    gxn = jnp.clip(2.0 * locx - 1.0, -1.0, 1.0)
    gyn = jnp.clip(2.0 * locy - 1.0, -1.0, 1.0)
    gx = ((gxn + 1.0) * Ws - 1.0) * 0.5
    gy = ((gyn + 1.0) * Hs - 1.0) * 0.5

    x0f = jnp.floor(gx)
    y0f = jnp.floor(gy)
    wx1 = gx - x0f
    wx0 = 1.0 - wx1
    wy1 = gy - y0f
    wy0 = 1.0 - wy1
    x0 = x0f.astype(jnp.int32)
    y0 = y0f.astype(jnp.int32)
    x1 = x0 + 1
    y1 = y0 + 1

    boff = b * P

    def corner(xc, yc, wxc, wyc):
        valid = ((xc >= 0) & (xc <= Ws - 1) & (yc >= 0) & (yc <= Hs - 1))
        idx = (jnp.clip(yc, 0, Hs - 1) * Ws + jnp.clip(xc, 0, Ws - 1) + boff)
        w = wxc * wyc * valid.astype(jnp.float32) * attn
        return idx, w

    i00, w00 = corner(x0, y0, wx0, wy0)
    i01, w01 = corner(x0, y1, wx0, wy1)
    i10, w10 = corner(x1, y0, wx1, wy0)
    i11, w11 = corner(x1, y1, wx1, wy1)

    i00_ref[0] = i00
    i01_ref[0] = i01
    i10_ref[0] = i10
    i11_ref[0] = i11
    w00_ref[0] = w00
    w01_ref[0] = w01
    w10_ref[0] = w10
    w11_ref[0] = w11


def _stage_a(queries, qgl, W_attn, b_attn, W_off_x, b_off_x, W_off_y, b_off_y, G):
    out_sd = jax.ShapeDtypeStruct((B, Q, HLK), jnp.int32)
    out_sw = jax.ShapeDtypeStruct((B, Q, HLK), jnp.float32)
    full = lambda shp: pl.BlockSpec(shp, lambda b: (0,) * len(shp))
    bq = pl.BlockSpec((1, Q, D), lambda b: (b, 0, 0))
    bg = pl.BlockSpec((1, Q, 4), lambda b: (b, 0, 0))
    bo = pl.BlockSpec((1, Q, HLK), lambda b: (b, 0, 0))
    return pl.pallas_call(
        _stage_a_body,
        grid=(B,),
        in_specs=[bq, bg, full((D, HLK)), full((1, HLK)), full((D, HLK)),
                  full((1, HLK)), full((D, HLK)), full((1, HLK)), full((HLK, HLK))],
        out_specs=[bo] * 8,
        out_shape=[out_sd] * 4 + [out_sw] * 4,
    )(queries, qgl, W_attn, b_attn, W_off_x, b_off_x, W_off_y, b_off_y, G)


def _vproj_body(vt_ref, w_ref, b_ref, out_ref):
    out_ref[...] = (jnp.dot(vt_ref[...], w_ref[...],
                            preferred_element_type=jnp.float32) + b_ref[...])


def _vproj(value_t, W_val, b_val):
    nb = 10
    rows = (B * P) // nb
    return pl.pallas_call(
        _vproj_body,
        grid=(nb,),
        in_specs=[pl.BlockSpec((rows, D), lambda i: (i, 0)),
                  pl.BlockSpec((D, Cv), lambda i: (0, 0)),
                  pl.BlockSpec((1, Cv), lambda i: (0, 0))],
        out_specs=pl.BlockSpec((rows, Cv), lambda i: (i, 0)),
        out_shape=jax.ShapeDtypeStruct((B * P, Cv), jnp.float32),
    )(value_t, W_val, b_val)


def _splat(wchunk, t):
    return lax.gather(
        wchunk, jnp.full((16, 1), t, jnp.int32),
        lax.GatherDimensionNumbers(offset_dims=(),
                                   collapsed_slice_dims=(0,),
                                   start_index_map=(0,)),
        (1,), mode=lax.GatherScatterMode.PROMISE_IN_BOUNDS)


CH = 128


def _sc_gather_body(table, idx, w, out, idx_c, w_c, rows0, rows1,
                    acc0, acc1, gsem, osem0, osem1):
    nc = 2
    wid = lax.axis_index("s") * nc + lax.axis_index("c")
    base = wid * PER_W

    def accumulate(rl, rows_v, acc_v):
        def jbody(jj, accs):
            wchunk = w_c[rl, pl.ds(jj * 16, 16)]
            for t in range(16):
                wv = _splat(wchunk, t)
                j = jj * 16 + t
                accs = tuple(accs[c] + wv * rows_v[j, pl.ds(c * 16, 16)]
                             for c in range(16))
            return accs

        accs = lax.fori_loop(
            0, (LK * 4) // 16, jbody,
            tuple(jnp.zeros((16,), jnp.float32) for _ in range(16)))
        for c in range(16):
            acc_v[pl.ds(c * 16, 16)] = accs[c]

    def issue_gather(rl, rows_v):
        pltpu.async_copy(table.at[idx_c.at[rl]], rows_v, gsem)

    def wait_gather(rl, rows_v):
        pltpu.make_async_copy(table.at[idx_c.at[rl]], rows_v, gsem).wait()

    def chunk_body(ci, _):
        gbase = base + ci * CH

        pltpu.sync_copy(idx.at[pl.ds(gbase, CH)], idx_c)
        pltpu.sync_copy(w.at[pl.ds(gbase, CH)], w_c)

        issue_gather(0, rows0)

        def body(i2, _):
            r0 = 2 * i2
            r1 = r0 + 1
            wait_gather(r0, rows0)
            issue_gather(r1, rows1)
            accumulate(r0, rows0, acc0)

            @pl.when(i2 >= 1)
            def _():
                pltpu.make_async_copy(acc0, out.at[gbase + r0 - 2],
                                      osem0).wait()
            pltpu.async_copy(acc0, out.at[gbase + r0], osem0)

            wait_gather(r1, rows1)

            @pl.when(i2 < CH // 2 - 1)
            def _():
                issue_gather(r1 + 1, rows0)
            accumulate(r1, rows1, acc1)

            @pl.when(i2 >= 1)
            def _():
                pltpu.make_async_copy(acc1, out.at[gbase + r1 - 2],
                                      osem1).wait()
            pltpu.async_copy(acc1, out.at[gbase + r1], osem1)
            return 0

        lax.fori_loop(0, CH // 2, body, 0)

        pltpu.make_async_copy(acc0, out.at[gbase + CH - 2], osem0).wait()
        pltpu.make_async_copy(acc1, out.at[gbase + CH - 1], osem1).wait()
        return 0

    lax.fori_loop(0, PER_W // CH, chunk_body, 0)


def _sc_gather(table, idx_rows, w_rows):
    mesh = plsc.VectorSubcoreMesh(core_axis_name="c", subcore_axis_name="s")
    kfn = functools.partial(
        pl.kernel, mesh=mesh,
        out_type=jax.ShapeDtypeStruct((NOUT_PAD, Cv), jnp.float32),
        scratch_types=[
            pltpu.VMEM((CH, LK * 4), jnp.int32),
            pltpu.VMEM((CH, LK * 4), jnp.float32),
            pltpu.VMEM((LK * 4, Cv), jnp.float32),
            pltpu.VMEM((LK * 4, Cv), jnp.float32),
            pltpu.VMEM((Cv,), jnp.float32),
            pltpu.VMEM((Cv,), jnp.float32),
            pltpu.SemaphoreType.DMA,
            pltpu.SemaphoreType.DMA,
            pltpu.SemaphoreType.DMA,
        ],
    )(_sc_gather_body)
    return kfn(table, idx_rows, w_rows)


def _final_body(hm_ref, wf_ref, bf_ref, out_ref):
    acc = jnp.dot(hm_ref[...], wf_ref[...], preferred_element_type=jnp.float32)
    out_ref[...] = acc + jnp.sum(bf_ref[...], axis=0, keepdims=True)


def _final(hm, Wf, b_final):
    return pl.pallas_call(
        _final_body,
        grid=(2,),
        in_specs=[pl.BlockSpec((B * Q // 2, H * Cv), lambda i: (i, 0)),
                  pl.BlockSpec((H * Cv, Cv), lambda i: (0, 0)),
                  pl.BlockSpec((H, Cv), lambda i: (0, 0))],
        out_specs=pl.BlockSpec((B * Q // 2, Cv), lambda i: (i, 0)),
        out_shape=jax.ShapeDtypeStruct((B * Q, Cv), jnp.float32),
    )(hm, Wf, b_final)


_G_BLOCK = np.kron(np.eye(H, dtype=np.float32), np.ones((LK, LK), np.float32))


def kernel(queries, query_geometry_logits, value, W_off, b_off, W_attn, b_attn,
           W_val, b_val, W_final, b_final):
    W_off_x = W_off[:, 0::2]
    W_off_y = W_off[:, 1::2]
    b_off_x = b_off[0::2].reshape(1, HLK)
    b_off_y = b_off[1::2].reshape(1, HLK)
    b_attn2 = b_attn.reshape(1, HLK)
    G = jnp.asarray(_G_BLOCK)

    i00, i01, i10, i11, w00, w01, w10, w11 = _stage_a(
        queries, query_geometry_logits, W_attn, b_attn2,
        W_off_x, b_off_x, W_off_y, b_off_y, G)

    idx_rows = jnp.stack([i00, i01, i10, i11], axis=-1).reshape(NOUT, LK * 4)
    w_rows = jnp.stack([w00, w01, w10, w11], axis=-1).reshape(NOUT, LK * 4)
    idx_rows = jnp.pad(idx_rows, ((0, NOUT_PAD - NOUT), (0, 0)))
    w_rows = jnp.pad(w_rows, ((0, NOUT_PAD - NOUT), (0, 0)))

    value_t = value.transpose(0, 2, 3, 1).reshape(B * P, D)
    table = _vproj(value_t, W_val, b_val.reshape(1, Cv))

    heads = _sc_gather(table, idx_rows, w_rows)[:NOUT]

    hm = heads.reshape(B * Q, H * Cv)
    Wf = W_final.reshape(H * Cv, Cv)
    out = _final(hm, Wf, b_final)
    return out.reshape(B, Q, Cv)


# speedup vs baseline: 13.1774x; 1.0549x over previous
import functools

import jax
import jax.numpy as jnp
import numpy as np
from jax import lax
from jax.experimental import pallas as pl
from jax.experimental.pallas import tpu as pltpu
from jax.experimental.pallas import tpu_sc as plsc

B, Q, D = 2, 1000, 256
H = 8
L = 4
K = 4
LK = L * K
Cv = 256
Hs, Ws = 100, 100
SCALE = 0.5
P = Hs * Ws
HLK = H * LK
NOUT = B * Q * H
NW = 32
NOUT_PAD = 16384
PER_W = NOUT_PAD // NW


def _stage_a_body(q_ref, qgl_ref, wattn_ref, battn_ref, woffx_ref, boffx_ref,
                  woffy_ref, boffy_ref, g_ref,
                  i00_ref, i01_ref, i10_ref, i11_ref,
                  w00_ref, w01_ref, w10_ref, w11_ref):
    b = pl.program_id(0)
    q = q_ref[0]

    attn_log = jnp.dot(q, wattn_ref[...], preferred_element_type=jnp.float32)
    attn_log = attn_log + battn_ref[...]
    m = jnp.max(attn_log, axis=-1, keepdims=True)
    e = jnp.exp(attn_log - m)
    s = jnp.dot(e, g_ref[...], preferred_element_type=jnp.float32)
    attn = e / s

    ox = jnp.dot(q, woffx_ref[...], preferred_element_type=jnp.float32) + boffx_ref[...]
    oy = jnp.dot(q, woffy_ref[...], preferred_element_type=jnp.float32) + boffy_ref[...]

    qgl = qgl_ref[0]
    box = 1.0 / (1.0 + jnp.exp(-qgl))
    cx = box[:, 0:1]
    cy = box[:, 1:2]
    sx = cx * (SCALE / LK)
    sy = cy * (SCALE / LK)

    locx = cx + ox * sx
    locy = cy + oy * sy
    gxn = jnp.clip(2.0 * locx - 1.0, -1.0, 1.0)
    gyn = jnp.clip(2.0 * locy - 1.0, -1.0, 1.0)
    gx = ((gxn + 1.0) * Ws - 1.0) * 0.5
    gy = ((gyn + 1.0) * Hs - 1.0) * 0.5

    x0f = jnp.floor(gx)
    y0f = jnp.floor(gy)
    wx1 = gx - x0f
    wx0 = 1.0 - wx1
    wy1 = gy - y0f
    wy0 = 1.0 - wy1
    x0 = x0f.astype(jnp.int32)
    y0 = y0f.astype(jnp.int32)
    x1 = x0 + 1
    y1 = y0 + 1

    boff = b * P

    def corner(xc, yc, wxc, wyc):
        valid = ((xc >= 0) & (xc <= Ws - 1) & (yc >= 0) & (yc <= Hs - 1))
        idx = (jnp.clip(yc, 0, Hs - 1) * Ws + jnp.clip(xc, 0, Ws - 1) + boff)
        w = wxc * wyc * valid.astype(jnp.float32) * attn
        return idx, w

    i00, w00 = corner(x0, y0, wx0, wy0)
    i01, w01 = corner(x0, y1, wx0, wy1)
    i10, w10 = corner(x1, y0, wx1, wy0)
    i11, w11 = corner(x1, y1, wx1, wy1)

    i00_ref[0] = i00
    i01_ref[0] = i01
    i10_ref[0] = i10
    i11_ref[0] = i11
    w00_ref[0] = w00
    w01_ref[0] = w01
    w10_ref[0] = w10
    w11_ref[0] = w11


def _stage_a(queries, qgl, W_attn, b_attn, W_off_x, b_off_x, W_off_y, b_off_y, G):
    out_sd = jax.ShapeDtypeStruct((B, Q, HLK), jnp.int32)
    out_sw = jax.ShapeDtypeStruct((B, Q, HLK), jnp.float32)
    full = lambda shp: pl.BlockSpec(shp, lambda b: (0,) * len(shp))
    bq = pl.BlockSpec((1, Q, D), lambda b: (b, 0, 0))
    bg = pl.BlockSpec((1, Q, 4), lambda b: (b, 0, 0))
    bo = pl.BlockSpec((1, Q, HLK), lambda b: (b, 0, 0))
    return pl.pallas_call(
        _stage_a_body,
        grid=(B,),
        in_specs=[bq, bg, full((D, HLK)), full((1, HLK)), full((D, HLK)),
                  full((1, HLK)), full((D, HLK)), full((1, HLK)), full((HLK, HLK))],
        out_specs=[bo] * 8,
        out_shape=[out_sd] * 4 + [out_sw] * 4,
    )(queries, qgl, W_attn, b_attn, W_off_x, b_off_x, W_off_y, b_off_y, G)


def _vproj_body(vt_ref, w_ref, b_ref, out_ref):
    out_ref[...] = (jnp.dot(vt_ref[...], w_ref[...],
                            preferred_element_type=jnp.float32) + b_ref[...])


def _vproj(value_t, W_val, b_val):
    nb = 10
    rows = (B * P) // nb
    return pl.pallas_call(
        _vproj_body,
        grid=(nb,),
        in_specs=[pl.BlockSpec((rows, D), lambda i: (i, 0)),
                  pl.BlockSpec((D, Cv), lambda i: (0, 0)),
                  pl.BlockSpec((1, Cv), lambda i: (0, 0))],
        out_specs=pl.BlockSpec((rows, Cv), lambda i: (i, 0)),
        out_shape=jax.ShapeDtypeStruct((B * P, Cv), jnp.float32),
    )(value_t, W_val, b_val)


def _splat(wchunk, t):
    return lax.gather(
        wchunk, jnp.full((16, 1), t, jnp.int32),
        lax.GatherDimensionNumbers(offset_dims=(),
                                   collapsed_slice_dims=(0,),
                                   start_index_map=(0,)),
        (1,), mode=lax.GatherScatterMode.PROMISE_IN_BOUNDS)


CH = 128


def _sc_gather_body(table, idx, w, out, idx_c, w_c, rows4,
                    acc0, acc1, gsem, osem0, osem1):
    nc = 2
    wid = lax.axis_index("s") * nc + lax.axis_index("c")
    base = wid * PER_W

    def accumulate(rl, rows_v, acc_v):
        def jbody(jj, accs):
            wchunk = w_c[rl, pl.ds(jj * 16, 16)]
            for t in range(16):
                wv = _splat(wchunk, t)
                j = jj * 16 + t
                accs = tuple(accs[c] + wv * rows_v[j, pl.ds(c * 16, 16)]
                             for c in range(16))
            return accs

        accs = lax.fori_loop(
            0, (LK * 4) // 16, jbody,
            tuple(jnp.zeros((16,), jnp.float32) for _ in range(16)))
        for c in range(16):
            acc_v[pl.ds(c * 16, 16)] = accs[c]

    def issue_gather(rl, rows_v):
        pltpu.async_copy(table.at[idx_c.at[rl]], rows_v, gsem)

    def wait_gather(rl, rows_v):
        pltpu.make_async_copy(table.at[idx_c.at[rl]], rows_v, gsem).wait()

    def chunk_body(ci, _):
        gbase = base + ci * CH

        pltpu.sync_copy(idx.at[pl.ds(gbase, CH)], idx_c)
        pltpu.sync_copy(w.at[pl.ds(gbase, CH)], w_c)

        for b in range(3):
            issue_gather(b, rows4.at[b])

        def body(i4, _):
            for b in range(4):
                r = 4 * i4 + b
                wait_gather(r, rows4.at[b])
                nxt = r + 3

                @pl.when(nxt < CH)
                def _():
                    issue_gather(nxt, rows4.at[(b + 3) % 4])

                acc_v = acc0 if b % 2 == 0 else acc1
                osem = osem0 if b % 2 == 0 else osem1
                accumulate(r, rows4.at[b], acc_v)

                @pl.when(r >= 2)
                def _():
                    pltpu.make_async_copy(acc_v, out.at[gbase + r - 2],
                                          osem).wait()
                pltpu.async_copy(acc_v, out.at[gbase + r], osem)
            return 0

        lax.fori_loop(0, CH // 4, body, 0)

        pltpu.make_async_copy(acc0, out.at[gbase + CH - 2], osem0).wait()
        pltpu.make_async_copy(acc1, out.at[gbase + CH - 1], osem1).wait()
        return 0

    lax.fori_loop(0, PER_W // CH, chunk_body, 0)


def _sc_gather(table, idx_rows, w_rows):
    mesh = plsc.VectorSubcoreMesh(core_axis_name="c", subcore_axis_name="s")
    kfn = functools.partial(
        pl.kernel, mesh=mesh,
        out_type=jax.ShapeDtypeStruct((NOUT_PAD, Cv), jnp.float32),
        scratch_types=[
            pltpu.VMEM((CH, LK * 4), jnp.int32),
            pltpu.VMEM((CH, LK * 4), jnp.float32),
            pltpu.VMEM((4, LK * 4, Cv), jnp.float32),
            pltpu.VMEM((Cv,), jnp.float32),
            pltpu.VMEM((Cv,), jnp.float32),
            pltpu.SemaphoreType.DMA,
            pltpu.SemaphoreType.DMA,
            pltpu.SemaphoreType.DMA,
        ],
    )(_sc_gather_body)
    return kfn(table, idx_rows, w_rows)


def _final_body(hm_ref, wf_ref, bf_ref, out_ref):
    acc = jnp.dot(hm_ref[...], wf_ref[...], preferred_element_type=jnp.float32)
    out_ref[...] = acc + jnp.sum(bf_ref[...], axis=0, keepdims=True)


def _final(hm, Wf, b_final):
    return pl.pallas_call(
        _final_body,
        grid=(2,),
        in_specs=[pl.BlockSpec((B * Q // 2, H * Cv), lambda i: (i, 0)),
                  pl.BlockSpec((H * Cv, Cv), lambda i: (0, 0)),
                  pl.BlockSpec((H, Cv), lambda i: (0, 0))],
        out_specs=pl.BlockSpec((B * Q // 2, Cv), lambda i: (i, 0)),
        out_shape=jax.ShapeDtypeStruct((B * Q, Cv), jnp.float32),
    )(hm, Wf, b_final)


_G_BLOCK = np.kron(np.eye(H, dtype=np.float32), np.ones((LK, LK), np.float32))


def kernel(queries, query_geometry_logits, value, W_off, b_off, W_attn, b_attn,
           W_val, b_val, W_final, b_final):
    W_off_x = W_off[:, 0::2]
    W_off_y = W_off[:, 1::2]
    b_off_x = b_off[0::2].reshape(1, HLK)
    b_off_y = b_off[1::2].reshape(1, HLK)
    b_attn2 = b_attn.reshape(1, HLK)
    G = jnp.asarray(_G_BLOCK)

    i00, i01, i10, i11, w00, w01, w10, w11 = _stage_a(
        queries, query_geometry_logits, W_attn, b_attn2,
        W_off_x, b_off_x, W_off_y, b_off_y, G)

    idx_rows = jnp.stack([i00, i01, i10, i11], axis=-1).reshape(NOUT, LK * 4)
    w_rows = jnp.stack([w00, w01, w10, w11], axis=-1).reshape(NOUT, LK * 4)
    idx_rows = jnp.pad(idx_rows, ((0, NOUT_PAD - NOUT), (0, 0)))
    w_rows = jnp.pad(w_rows, ((0, NOUT_PAD - NOUT), (0, 0)))

    value_t = value.transpose(0, 2, 3, 1).reshape(B * P, D)
    table = _vproj(value_t, W_val, b_val.reshape(1, Cv))

    heads = _sc_gather(table, idx_rows, w_rows)[:NOUT]

    hm = heads.reshape(B * Q, H * Cv)
    Wf = W_final.reshape(H * Cv, Cv)
    out = _final(hm, Wf, b_final)
    return out.reshape(B, Q, Cv)
